# baseline (speedup 1.0000x reference)
import functools

import jax
import jax.numpy as jnp
from jax import lax
from jax.experimental import pallas as pl
from jax.experimental.pallas import tpu as pltpu
from jax.experimental.pallas import tpu_sc as plsc

N = 50000
D = 512
S = 128
NW = 32

C = 40
N_SC = 25600
NCHUNK = N_SC // C
TRIPS = NCHUNK // NW
PAIRS = (TRIPS + 1) // 2
IDXROWS = TRIPS * C
NV = D // 16

BR = 400
NB_ALL = N // BR
NB_SKIP = N_SC // BR
NB_TC = NB_ALL - NB_SKIP


def _sc_partial(x, edge):
    mesh = plsc.VectorSubcoreMesh(core_axis_name="c", subcore_axis_name="s")

    @functools.partial(
        pl.kernel,
        mesh=mesh,
        out_type=jax.ShapeDtypeStruct((NW, S, D), jnp.float32),
        scratch_types=[
            pltpu.VMEM((C, D), jnp.float32),
            pltpu.VMEM((C, D), jnp.float32),
            pltpu.VMEM((IDXROWS + 16,), jnp.int32),
            pltpu.VMEM((S + 1, D), jnp.float32),
            pltpu.VMEM((D,), jnp.float32),
            pltpu.SemaphoreType.DMA,
            pltpu.SemaphoreType.DMA,
        ],
    )
    def body(x_hbm, e_hbm, out_hbm, buf0, buf1, idx_v, part_v, acc_v,
             sem0, sem1):
        cid = lax.axis_index("c")
        sid = lax.axis_index("s")
        w = sid * 2 + cid

        start = w * TRIPS

        def dma(jj, buf, sem):
            return pltpu.make_async_copy(
                x_hbm.at[pl.ds((start + jj) * C, C)], buf, sem)

        dma(0, buf0, sem0).start()

        pltpu.sync_copy(e_hbm.at[pl.ds(start * C, IDXROWS)],
                        idx_v.at[pl.ds(0, IDXROWS)])

        z16 = jnp.zeros((16,), jnp.float32)

        def zero_row(r, _):
            def zero_vec(k, _):
                part_v[r, pl.ds(k * 16, 16)] = z16
                return 0
            return lax.fori_loop(0, NV, zero_vec, 0)

        lax.fori_loop(0, S + 1, zero_row, 0)
        for k in range(NV):
            acc_v[pl.ds(k * 16, 16)] = z16

        def flush(seg):
            for k in range(NV):
                v = acc_v[pl.ds(k * 16, 16)]
                plsc.addupdate(part_v.at[seg, pl.ds(k * 16, 16)], v)
                acc_v[pl.ds(k * 16, 16)] = z16

        def process(jj, rows_v, run_seg):
            pos = jj * C
            seg0 = idx_v[pl.ds(pos, 16)][0]
            seglast = idx_v[pl.ds(pos + C - 16, 16)][15]
            uniform = seg0 == seglast

            def hot(rs):
                for k in range(NV):
                    ss = [rows_v[i, pl.ds(k * 16, 16)] for i in range(4)]
                    for i in range(4, C):
                        ss[i % 4] = ss[i % 4] + rows_v[i, pl.ds(k * 16, 16)]
                    t = (ss[0] + ss[1]) + (ss[2] + ss[3])
                    plsc.addupdate(acc_v.at[pl.ds(k * 16, 16)], t)
                return rs

            def slow(rs):
                def row(i, r):
                    seg = idx_v[pl.ds(pos + i, 16)][0]

                    def chg(_):
                        flush(r)
                        return seg

                    r = lax.cond(seg != r, chg, lambda rr: rr, r)
                    for k in range(NV):
                        plsc.addupdate(
                            acc_v.at[pl.ds(k * 16, 16)],
                            rows_v[i, pl.ds(k * 16, 16)],
                        )
                    return r

                return lax.fori_loop(0, C, row, rs)

            return lax.cond(uniform & (seg0 == run_seg), hot, slow, run_seg)

        def pair(j, run_seg):
            jj0 = 2 * j
            jj1 = jj0 + 1
            jj2 = jj0 + 2

            @pl.when(jj1 < TRIPS)
            def _():
                dma(jj1, buf1, sem1).start()

            dma(jj0, buf0, sem0).wait()
            run_seg = process(jj0, buf0, run_seg)

            @pl.when(jj2 < TRIPS)
            def _():
                dma(jj2, buf0, sem0).start()

            def p1(rs):
                dma(jj1, buf1, sem1).wait()
                return process(jj1, buf1, rs)

            return lax.cond(jj1 < TRIPS, p1, lambda rs: rs, run_seg)

        run_seg = lax.fori_loop(0, PAIRS, pair, jnp.int32(S))
        flush(run_seg)

        pltpu.sync_copy(part_v.at[pl.ds(0, S)], out_hbm.at[w])

    return body(x, edge)


def _tc_seg_body(seg_ref, x_ref, o_ref):
    j = pl.program_id(0)
    segc = jnp.reshape(seg_ref[...], (BR, 1))
    oh = (segc == lax.broadcasted_iota(jnp.int32, (1, S), 1)).astype(
        jnp.float32)
    part = jax.lax.dot_general(oh, x_ref[...], (((0,), (0,)), ((), ())),
                               preferred_element_type=jnp.float32)

    @pl.when(j == 0)
    def _():
        o_ref[...] = jnp.zeros_like(o_ref)

    o_ref[...] += part


def _tc_partial(x, e32):
    e3 = e32.reshape(NB_ALL, 1, BR)
    return pl.pallas_call(
        _tc_seg_body,
        grid=(NB_TC,),
        in_specs=[
            pl.BlockSpec((1, 1, BR), lambda j: (j + NB_SKIP, 0, 0)),
            pl.BlockSpec((BR, D), lambda j: (j + NB_SKIP, 0)),
        ],
        out_specs=pl.BlockSpec((S, D), lambda j: (0, 0)),
        out_shape=jax.ShapeDtypeStruct((S, D), jnp.float32),
    )(e3, x)


def _combine_body(p_ref, t_ref, o_ref):
    o_ref[...] = jnp.sum(p_ref[...], axis=0) + t_ref[...]


def kernel(x, edge_list):
    e32 = edge_list.astype(jnp.int32)
    sc_part = _sc_partial(x, e32)
    tc_part = _tc_partial(x, e32)
    return pl.pallas_call(
        _combine_body,
        out_shape=jax.ShapeDtypeStruct((S, D), jnp.float32),
    )(sc_part, tc_part)


# speedup vs baseline: 1.1416x; 1.1416x over previous
import functools

import jax
import jax.numpy as jnp
from jax import lax
from jax.experimental import pallas as pl
from jax.experimental.pallas import tpu as pltpu
from jax.experimental.pallas import tpu_sc as plsc

N = 50000
D = 512
S = 128
NW = 32

C = 40
N_SC = 19200
NCHUNK = N_SC // C
TRIPS = NCHUNK // NW
PAIRS = (TRIPS + 1) // 2
IDXROWS = TRIPS * C
NV = D // 16

BR = 400
NB_ALL = N // BR
NB_SKIP = N_SC // BR
NB_TC = NB_ALL - NB_SKIP


def _sc_partial(x, edge):
    mesh = plsc.VectorSubcoreMesh(core_axis_name="c", subcore_axis_name="s")

    @functools.partial(
        pl.kernel,
        mesh=mesh,
        out_type=jax.ShapeDtypeStruct((NW, S, D), jnp.float32),
        scratch_types=[
            pltpu.VMEM((C, D), jnp.float32),
            pltpu.VMEM((C, D), jnp.float32),
            pltpu.VMEM((IDXROWS + 16,), jnp.int32),
            pltpu.VMEM((S + 1, D), jnp.float32),
            pltpu.VMEM((D,), jnp.float32),
            pltpu.SemaphoreType.DMA,
            pltpu.SemaphoreType.DMA,
        ],
    )
    def body(x_hbm, e_hbm, out_hbm, buf0, buf1, idx_v, part_v, acc_v,
             sem0, sem1):
        cid = lax.axis_index("c")
        sid = lax.axis_index("s")
        w = sid * 2 + cid

        start = w * TRIPS

        def dma(jj, buf, sem):
            return pltpu.make_async_copy(
                x_hbm.at[pl.ds((start + jj) * C, C)], buf, sem)

        dma(0, buf0, sem0).start()

        pltpu.sync_copy(e_hbm.at[pl.ds(start * C, IDXROWS)],
                        idx_v.at[pl.ds(0, IDXROWS)])

        z16 = jnp.zeros((16,), jnp.float32)

        def zero_row(r, _):
            def zero_vec(k, _):
                part_v[r, pl.ds(k * 16, 16)] = z16
                return 0
            return lax.fori_loop(0, NV, zero_vec, 0)

        lax.fori_loop(0, S + 1, zero_row, 0)
        for k in range(NV):
            acc_v[pl.ds(k * 16, 16)] = z16

        def flush(seg):
            for k in range(NV):
                v = acc_v[pl.ds(k * 16, 16)]
                plsc.addupdate(part_v.at[seg, pl.ds(k * 16, 16)], v)
                acc_v[pl.ds(k * 16, 16)] = z16

        def process(jj, rows_v, run_seg):
            pos = jj * C
            seg0 = idx_v[pl.ds(pos, 16)][0]
            seglast = idx_v[pl.ds(pos + C - 16, 16)][15]
            uniform = seg0 == seglast

            def hot(rs):
                for k in range(NV):
                    ss = [rows_v[i, pl.ds(k * 16, 16)] for i in range(4)]
                    for i in range(4, C):
                        ss[i % 4] = ss[i % 4] + rows_v[i, pl.ds(k * 16, 16)]
                    t = (ss[0] + ss[1]) + (ss[2] + ss[3])
                    plsc.addupdate(acc_v.at[pl.ds(k * 16, 16)], t)
                return rs

            def slow(rs):
                def row(i, r):
                    seg = idx_v[pl.ds(pos + i, 16)][0]

                    def chg(_):
                        flush(r)
                        return seg

                    r = lax.cond(seg != r, chg, lambda rr: rr, r)
                    for k in range(NV):
                        plsc.addupdate(
                            acc_v.at[pl.ds(k * 16, 16)],
                            rows_v[i, pl.ds(k * 16, 16)],
                        )
                    return r

                return lax.fori_loop(0, C, row, rs)

            return lax.cond(uniform & (seg0 == run_seg), hot, slow, run_seg)

        def pair(j, run_seg):
            jj0 = 2 * j
            jj1 = jj0 + 1
            jj2 = jj0 + 2

            @pl.when(jj1 < TRIPS)
            def _():
                dma(jj1, buf1, sem1).start()

            dma(jj0, buf0, sem0).wait()
            run_seg = process(jj0, buf0, run_seg)

            @pl.when(jj2 < TRIPS)
            def _():
                dma(jj2, buf0, sem0).start()

            def p1(rs):
                dma(jj1, buf1, sem1).wait()
                return process(jj1, buf1, rs)

            return lax.cond(jj1 < TRIPS, p1, lambda rs: rs, run_seg)

        run_seg = lax.fori_loop(0, PAIRS, pair, jnp.int32(S))
        flush(run_seg)

        pltpu.sync_copy(part_v.at[pl.ds(0, S)], out_hbm.at[w])

    return body(x, edge)


def _tc_seg_body(seg_ref, x_ref, o_ref):
    j = pl.program_id(0)
    segc = jnp.reshape(seg_ref[...], (BR, 1))
    oh = (segc == lax.broadcasted_iota(jnp.int32, (1, S), 1)).astype(
        jnp.float32)
    part = jax.lax.dot_general(oh, x_ref[...], (((0,), (0,)), ((), ())),
                               preferred_element_type=jnp.float32)

    @pl.when(j == 0)
    def _():
        o_ref[...] = jnp.zeros_like(o_ref)

    o_ref[...] += part


def _tc_partial(x, e32):
    e3 = e32.reshape(NB_ALL, 1, BR)
    return pl.pallas_call(
        _tc_seg_body,
        grid=(NB_TC,),
        in_specs=[
            pl.BlockSpec((1, 1, BR), lambda j: (j + NB_SKIP, 0, 0)),
            pl.BlockSpec((BR, D), lambda j: (j + NB_SKIP, 0)),
        ],
        out_specs=pl.BlockSpec((S, D), lambda j: (0, 0)),
        out_shape=jax.ShapeDtypeStruct((S, D), jnp.float32),
    )(e3, x)


def _combine_body(p_ref, t_ref, o_ref):
    o_ref[...] = jnp.sum(p_ref[...], axis=0) + t_ref[...]


def kernel(x, edge_list):
    e32 = edge_list.astype(jnp.int32)
    sc_part = _sc_partial(x, e32)
    tc_part = _tc_partial(x, e32)
    return pl.pallas_call(
        _combine_body,
        out_shape=jax.ShapeDtypeStruct((S, D), jnp.float32),
    )(sc_part, tc_part)
